# speedup vs baseline: 21.1410x; 21.1410x over previous
import jax
import jax.numpy as jnp
from jax.experimental import pallas as pl
from jax.experimental.pallas import tpu as pltpu

_A = 9
_H = 50
_W = 50
_N = _H * _W * _A
_ROWS = 192
_NPAD = _ROWS * 128
_PRE = 6000
_POST = 2000
_OUT_ROWS = 2048


def _rpn_body(l0, l1, d0, d1, d2, d3, a0, a1, a2, a3,
              out_ref, s_ref, x1_ref, y1_ref, x2_ref, y2_ref, ar_ref,
              live_ref):
    f32 = jnp.float32
    pos = (jax.lax.broadcasted_iota(jnp.int32, (_ROWS, 128), 0) * 128
           + jax.lax.broadcasted_iota(jnp.int32, (_ROWS, 128), 1))
    real = pos < _N

    l0v = l0[...]
    l1v = l1[...]
    mx = jnp.maximum(l0v, l1v)
    e0 = jnp.exp(l0v - mx)
    e1 = jnp.exp(l1v - mx)
    s = e1 / (e0 + e1)
    s_ref[...] = s

    w = a2[...] - a0[...]
    h = a3[...] - a1[...]
    cx = a0[...] + 0.5 * w
    cy = a1[...] + 0.5 * h
    pcx = d0[...] * w + cx
    pcy = d1[...] * h + cy
    pw = jnp.exp(jnp.clip(d2[...], -4.0, 4.0)) * w
    ph = jnp.exp(jnp.clip(d3[...], -4.0, 4.0)) * h
    x1 = pcx - 0.5 * pw
    y1 = pcy - 0.5 * ph
    x2 = pcx + 0.5 * pw
    y2 = pcy + 0.5 * ph
    x1_ref[...] = x1
    y1_ref[...] = y1
    x2_ref[...] = x2
    y2_ref[...] = y2
    ar_ref[...] = (x2 - x1) * (y2 - y1)

    key = jnp.where(real, jax.lax.bitcast_convert_type(s, jnp.int32),
                    jnp.int32(-1))
    vstar = jnp.int32(0)
    for b in range(30, -1, -1):
        t = vstar | jnp.int32(1 << b)
        cnt = jnp.sum((key >= t).astype(jnp.int32))
        vstar = jnp.where(cnt >= _PRE, t, vstar)
    n_gt = jnp.sum((key > vstar).astype(jnp.int32))
    need = _PRE - n_gt
    tie = key == vstar
    cc = jnp.int32(0)
    for b in range(14, -1, -1):
        t = cc | jnp.int32(1 << b)
        cnt = jnp.sum((tie & (pos < t)).astype(jnp.int32))
        cc = jnp.where(cnt < need, t, cc)
    cstar = jnp.where(need > 0, cc + 1, jnp.int32(0))
    live_ref[...] = ((key > vstar) | (tie & (pos < cstar))).astype(f32)

    out_ref[...] = jnp.zeros((_OUT_ROWS, 128), f32)
    ci = jax.lax.broadcasted_iota(jnp.int32, (1, 128), 1)

    def body(k, carry):
        live = live_ref[...]
        ls = jnp.where(live > 0.0, s_ref[...], -1.0)
        m = jnp.max(ls)
        valid = m >= 0.0

        @pl.when(valid)
        def _():
            eq = ls == m
            bi = jnp.min(jnp.where(eq, pos, jnp.int32(_NPAD)))
            selm = pos == bi
            sf = selm.astype(f32)
            bx1 = jnp.sum(x1_ref[...] * sf)
            by1 = jnp.sum(y1_ref[...] * sf)
            bx2 = jnp.sum(x2_ref[...] * sf)
            by2 = jnp.sum(y2_ref[...] * sf)
            bar = jnp.sum(ar_ref[...] * sf)
            xx1 = jnp.maximum(bx1, x1_ref[...])
            yy1 = jnp.maximum(by1, y1_ref[...])
            xx2 = jnp.minimum(bx2, x2_ref[...])
            yy2 = jnp.minimum(by2, y2_ref[...])
            inter = (jnp.maximum(xx2 - xx1, 0.0)
                     * jnp.maximum(yy2 - yy1, 0.0))
            iou = inter / (bar + ar_ref[...] - inter + 1e-9)
            supp = (iou > 0.7) | selm
            live_ref[...] = jnp.where(supp, 0.0, live)
            row = jnp.where(ci == 0, bx1,
                            jnp.where(ci == 1, by1,
                                      jnp.where(ci == 2, bx2,
                                                jnp.where(ci == 3, by2,
                                                          0.0))))
            out_ref[pl.ds(k, 1), :] = row
        return carry

    jax.lax.fori_loop(0, _POST, body, jnp.int32(0))


def _prep(v):
    flat = v.T.reshape(-1)
    return jnp.pad(flat, (0, _NPAD - _N)).reshape(_ROWS, 128)


def kernel(cls_logits, bbox_preds, anchors):
    cl = cls_logits.reshape(_A, 2, _H * _W)
    bb = bbox_preds.reshape(_A, 4, _H * _W)
    l0 = _prep(cl[:, 0, :])
    l1 = _prep(cl[:, 1, :])
    d = [_prep(bb[:, k, :]) for k in range(4)]
    a = [jnp.pad(anchors[:, k], (0, _NPAD - _N)).reshape(_ROWS, 128)
         for k in range(4)]
    out = pl.pallas_call(
        _rpn_body,
        out_shape=jax.ShapeDtypeStruct((_OUT_ROWS, 128), jnp.float32),
        scratch_shapes=[pltpu.VMEM((_ROWS, 128), jnp.float32)
                        for _ in range(7)],
    )(l0, l1, *d, *a)
    return out[:_POST, :4].reshape(1, _POST, 4)


# speedup vs baseline: 22.1969x; 1.0499x over previous
import functools

import jax
import jax.numpy as jnp
from jax import lax
from jax.experimental import pallas as pl
from jax.experimental.pallas import tpu as pltpu
from jax.experimental.pallas import tpu_sc as plsc

_A = 9
_H = 50
_W = 50
_N = _H * _W * _A
_ROWS = 192
_NPAD = _ROWS * 128
_PRE = 6000
_POST = 2000
_NSEL = 6144
_SROWS = 48
_TRASH = 6100
_OUT_ROWS = 2048
_NTILES = 32
_CHUNK = _NPAD // _NTILES


def _stage1_body(l0, l1, d0, d1, d2, d3, a0, a1, a2, a3,
                 s_o, x1_o, y1_o, x2_o, y2_o, dest_o):
    f32 = jnp.float32
    pos = (lax.broadcasted_iota(jnp.int32, (_ROWS, 128), 0) * 128
           + lax.broadcasted_iota(jnp.int32, (_ROWS, 128), 1))
    real = pos < _N

    l0v = l0[...]
    l1v = l1[...]
    mx = jnp.maximum(l0v, l1v)
    e0 = jnp.exp(l0v - mx)
    e1 = jnp.exp(l1v - mx)
    s = e1 / (e0 + e1)
    s_o[...] = s

    w = a2[...] - a0[...]
    h = a3[...] - a1[...]
    cx = a0[...] + 0.5 * w
    cy = a1[...] + 0.5 * h
    pcx = d0[...] * w + cx
    pcy = d1[...] * h + cy
    pw = jnp.exp(jnp.clip(d2[...], -4.0, 4.0)) * w
    ph = jnp.exp(jnp.clip(d3[...], -4.0, 4.0)) * h
    x1_o[...] = pcx - 0.5 * pw
    y1_o[...] = pcy - 0.5 * ph
    x2_o[...] = pcx + 0.5 * pw
    y2_o[...] = pcy + 0.5 * ph

    key = jnp.where(real, lax.bitcast_convert_type(s, jnp.int32),
                    jnp.int32(-1))
    vstar = jnp.int32(0)
    for b in range(30, -1, -1):
        t = vstar | jnp.int32(1 << b)
        cnt = jnp.sum((key >= t).astype(jnp.int32))
        vstar = jnp.where(cnt >= _PRE, t, vstar)
    n_gt = jnp.sum((key > vstar).astype(jnp.int32))
    need = _PRE - n_gt
    tie = key == vstar
    cc = jnp.int32(0)
    for b in range(14, -1, -1):
        t = cc | jnp.int32(1 << b)
        cnt = jnp.sum((tie & (pos < t)).astype(jnp.int32))
        cc = jnp.where(cnt < need, t, cc)
    cstar = jnp.where(need > 0, cc + 1, jnp.int32(0))
    mask = (key > vstar) | (tie & (pos < cstar))

    mf = mask.astype(f32)
    tri_u = (lax.broadcasted_iota(jnp.int32, (128, 128), 0)
             <= lax.broadcasted_iota(jnp.int32, (128, 128), 1)).astype(f32)
    rowcum = jnp.dot(mf, tri_u, preferred_element_type=f32)
    totals = lax.broadcast_in_dim(rowcum[:, 127], (_ROWS, 128), (0,))
    tri_l = (lax.broadcasted_iota(jnp.int32, (_ROWS, _ROWS), 1)
             < lax.broadcasted_iota(jnp.int32, (_ROWS, _ROWS), 0)).astype(f32)
    offs = jnp.dot(tri_l, totals, preferred_element_type=f32)
    cum = rowcum + offs
    dest_o[...] = jnp.where(mask, cum.astype(jnp.int32) - 1,
                            jnp.int32(_TRASH))


def _stage3_body(dat_ref, out_ref, live_ref):
    f32 = jnp.float32
    s = dat_ref[0]
    x1 = dat_ref[1]
    y1 = dat_ref[2]
    x2 = dat_ref[3]
    y2 = dat_ref[4]
    area = (x2 - x1) * (y2 - y1)
    pos = (lax.broadcasted_iota(jnp.int32, (_SROWS, 128), 0) * 128
           + lax.broadcasted_iota(jnp.int32, (_SROWS, 128), 1))
    live_ref[...] = (pos < _PRE).astype(f32)
    out_ref[...] = jnp.zeros((_OUT_ROWS, 128), f32)
    ci = lax.broadcasted_iota(jnp.int32, (1, 128), 1)

    def body(k, carry):
        live = live_ref[...]
        ls = jnp.where(live > 0.0, s, -1.0)
        m = jnp.max(ls)
        valid = m >= 0.0

        @pl.when(valid)
        def _():
            eq = ls == m
            bi = jnp.min(jnp.where(eq, pos, jnp.int32(_NSEL)))
            selm = pos == bi
            bx1 = jnp.sum(jnp.where(selm, x1, 0.0))
            by1 = jnp.sum(jnp.where(selm, y1, 0.0))
            bx2 = jnp.sum(jnp.where(selm, x2, 0.0))
            by2 = jnp.sum(jnp.where(selm, y2, 0.0))
            bar = jnp.sum(jnp.where(selm, area, 0.0))
            xx1 = jnp.maximum(bx1, x1)
            yy1 = jnp.maximum(by1, y1)
            xx2 = jnp.minimum(bx2, x2)
            yy2 = jnp.minimum(by2, y2)
            inter = (jnp.maximum(xx2 - xx1, 0.0)
                     * jnp.maximum(yy2 - yy1, 0.0))
            iou = inter / (bar + area - inter + 1e-9)
            supp = (iou > 0.7) | selm
            live_ref[...] = jnp.where(supp, 0.0, live)
            row = jnp.where(ci == 0, bx1,
                            jnp.where(ci == 1, by1,
                                      jnp.where(ci == 2, bx2,
                                                jnp.where(ci == 3, by2,
                                                          0.0))))
            out_ref[pl.ds(k, 1), :] = row
        return carry

    lax.fori_loop(0, _POST, body, jnp.int32(0))


def _sc_compact_call(dest3, packed):
    mesh = plsc.VectorSubcoreMesh(core_axis_name="c", subcore_axis_name="s")

    @functools.partial(
        pl.kernel,
        mesh=mesh,
        out_type=jax.ShapeDtypeStruct((_NSEL, 16), jnp.float32),
        compiler_params=pltpu.CompilerParams(use_tc_tiling_on_sc=False),
        scratch_types=[
            pltpu.VMEM((_CHUNK // 128, 128), jnp.int32),
            pltpu.VMEM((_CHUNK, 16), jnp.float32),
            pltpu.SemaphoreType.DMA,
        ],
    )
    def sc_compact(dest_hbm, data_hbm, out_hbm, idx_v, data_v, sem):
        wid = lax.axis_index("s") * 2 + lax.axis_index("c")
        base = wid * _CHUNK
        pltpu.sync_copy(dest_hbm.at[wid], idx_v)
        pltpu.sync_copy(data_hbm.at[pl.ds(base, _CHUNK)], data_v)
        descs = []
        for j in range(_CHUNK // 128):
            descs.append(pltpu.async_copy(
                data_v.at[pl.ds(j * 128, 128)],
                out_hbm.at[idx_v.at[j]],
                sem))
        for d in descs:
            d.wait()

    return sc_compact(dest3, packed)


def _prep(v):
    flat = v.T.reshape(-1)
    return jnp.pad(flat, (0, _NPAD - _N)).reshape(_ROWS, 128)


def kernel(cls_logits, bbox_preds, anchors):
    cl = cls_logits.reshape(_A, 2, _H * _W)
    bb = bbox_preds.reshape(_A, 4, _H * _W)
    l0 = _prep(cl[:, 0, :])
    l1 = _prep(cl[:, 1, :])
    d = [_prep(bb[:, k, :]) for k in range(4)]
    a = [jnp.pad(anchors[:, k], (0, _NPAD - _N)).reshape(_ROWS, 128)
         for k in range(4)]

    f32 = jnp.float32
    s, x1, y1, x2, y2, dest = pl.pallas_call(
        _stage1_body,
        out_shape=(
            jax.ShapeDtypeStruct((_ROWS, 128), f32),
            jax.ShapeDtypeStruct((_ROWS, 128), f32),
            jax.ShapeDtypeStruct((_ROWS, 128), f32),
            jax.ShapeDtypeStruct((_ROWS, 128), f32),
            jax.ShapeDtypeStruct((_ROWS, 128), f32),
            jax.ShapeDtypeStruct((_ROWS, 128), jnp.int32),
        ),
    )(l0, l1, *d, *a)

    packed = jnp.stack(
        [v.reshape(-1) for v in (s, x1, y1, x2, y2)], axis=-1)
    packed = jnp.pad(packed, ((0, 0), (0, 11)))
    dest3 = dest.reshape(_NTILES, _CHUNK // 128, 128)

    compact = _sc_compact_call(dest3, packed)

    dat = compact.T.reshape(16, _SROWS, 128)
    out = pl.pallas_call(
        _stage3_body,
        out_shape=jax.ShapeDtypeStruct((_OUT_ROWS, 128), f32),
        scratch_shapes=[pltpu.VMEM((_SROWS, 128), f32)],
    )(dat)
    return out[:_POST, :4].reshape(1, _POST, 4)


# speedup vs baseline: 27.9134x; 1.2575x over previous
import functools

import jax
import jax.numpy as jnp
from jax import lax
from jax.experimental import pallas as pl
from jax.experimental.pallas import tpu as pltpu
from jax.experimental.pallas import tpu_sc as plsc

_A = 9
_H = 50
_W = 50
_N = _H * _W * _A
_ROWS = 192
_NPAD = _ROWS * 128
_PRE = 6000
_POST = 2000
_NSEL = 6144
_SROWS = 48
_TRASH = 6100
_OUT_ROWS = 2048
_NTILES = 32
_CHUNK = _NPAD // _NTILES


def _stage1_body(l0, l1, d0, d1, d2, d3, a0, a1, a2, a3,
                 s_o, x1_o, y1_o, x2_o, y2_o, ar_o, dest_o):
    f32 = jnp.float32
    pos = (lax.broadcasted_iota(jnp.int32, (_ROWS, 128), 0) * 128
           + lax.broadcasted_iota(jnp.int32, (_ROWS, 128), 1))
    real = pos < _N

    l0v = l0[...]
    l1v = l1[...]
    mx = jnp.maximum(l0v, l1v)
    e0 = jnp.exp(l0v - mx)
    e1 = jnp.exp(l1v - mx)
    s = e1 / (e0 + e1)
    s_o[...] = s

    w = a2[...] - a0[...]
    h = a3[...] - a1[...]
    cx = a0[...] + 0.5 * w
    cy = a1[...] + 0.5 * h
    pcx = d0[...] * w + cx
    pcy = d1[...] * h + cy
    pw = jnp.exp(jnp.clip(d2[...], -4.0, 4.0)) * w
    ph = jnp.exp(jnp.clip(d3[...], -4.0, 4.0)) * h
    x1 = pcx - 0.5 * pw
    y1 = pcy - 0.5 * ph
    x2 = pcx + 0.5 * pw
    y2 = pcy + 0.5 * ph
    x1_o[...] = x1
    y1_o[...] = y1
    x2_o[...] = x2
    y2_o[...] = y2
    ar_o[...] = (x2 - x1) * (y2 - y1)

    key = jnp.where(real, lax.bitcast_convert_type(s, jnp.int32),
                    jnp.int32(-1))
    vstar = jnp.int32(0)
    for b in range(30, -1, -1):
        t = vstar | jnp.int32(1 << b)
        cnt = jnp.sum((key >= t).astype(jnp.int32))
        vstar = jnp.where(cnt >= _PRE, t, vstar)
    n_gt = jnp.sum((key > vstar).astype(jnp.int32))
    need = _PRE - n_gt
    tie = key == vstar
    cc = jnp.int32(0)
    for b in range(14, -1, -1):
        t = cc | jnp.int32(1 << b)
        cnt = jnp.sum((tie & (pos < t)).astype(jnp.int32))
        cc = jnp.where(cnt < need, t, cc)
    cstar = jnp.where(need > 0, cc + 1, jnp.int32(0))
    mask = (key > vstar) | (tie & (pos < cstar))

    mf = mask.astype(f32)
    tri_u = (lax.broadcasted_iota(jnp.int32, (128, 128), 0)
             <= lax.broadcasted_iota(jnp.int32, (128, 128), 1)).astype(f32)
    rowcum = jnp.dot(mf, tri_u, preferred_element_type=f32)
    totals = lax.broadcast_in_dim(rowcum[:, 127], (_ROWS, 128), (0,))
    tri_l = (lax.broadcasted_iota(jnp.int32, (_ROWS, _ROWS), 1)
             < lax.broadcasted_iota(jnp.int32, (_ROWS, _ROWS), 0)).astype(f32)
    offs = jnp.dot(tri_l, totals, preferred_element_type=f32)
    cum = rowcum + offs
    dest_o[...] = jnp.where(mask, cum.astype(jnp.int32) - 1,
                            jnp.int32(_TRASH))


def _stage3_body(dat_ref, x1s, y1s, x2s, y2s, ars, out_ref, ls_ref):
    f32 = jnp.float32
    s = dat_ref[0]
    x1 = dat_ref[1]
    y1 = dat_ref[2]
    x2 = dat_ref[3]
    y2 = dat_ref[4]
    area = dat_ref[5]
    pos = (lax.broadcasted_iota(jnp.int32, (_SROWS, 128), 0) * 128
           + lax.broadcasted_iota(jnp.int32, (_SROWS, 128), 1))
    ls_ref[...] = jnp.where(pos < _PRE, s, -1.0)
    out_ref[...] = jnp.zeros((_OUT_ROWS, 128), f32)
    ci = lax.broadcasted_iota(jnp.int32, (1, 128), 1)

    def body(k, carry):
        ls = ls_ref[...]
        m = jnp.max(ls)
        valid = m >= 0.0

        @pl.when(valid)
        def _():
            eq = ls == m
            bi = jnp.min(jnp.where(eq, pos, jnp.int32(_NSEL)))
            r = lax.div(bi, jnp.int32(128))
            c = lax.rem(bi, jnp.int32(128))
            bx1 = x1s[r, c]
            by1 = y1s[r, c]
            bx2 = x2s[r, c]
            by2 = y2s[r, c]
            bar = ars[r, c]
            xx1 = jnp.maximum(bx1, x1)
            yy1 = jnp.maximum(by1, y1)
            xx2 = jnp.minimum(bx2, x2)
            yy2 = jnp.minimum(by2, y2)
            inter = (jnp.maximum(xx2 - xx1, 0.0)
                     * jnp.maximum(yy2 - yy1, 0.0))
            iou = inter / (bar + area - inter + 1e-9)
            supp = (iou > 0.7) | (pos == bi)
            ls_ref[...] = jnp.where(supp, -1.0, ls)
            row = jnp.where(ci == 0, bx1,
                            jnp.where(ci == 1, by1,
                                      jnp.where(ci == 2, bx2,
                                                jnp.where(ci == 3, by2,
                                                          0.0))))
            out_ref[pl.ds(k, 1), :] = row
        return carry

    lax.fori_loop(0, _POST, body, jnp.int32(0))


def _sc_compact_call(dest3, packed):
    mesh = plsc.VectorSubcoreMesh(core_axis_name="c", subcore_axis_name="s")

    @functools.partial(
        pl.kernel,
        mesh=mesh,
        out_type=jax.ShapeDtypeStruct((_NSEL, 16), jnp.float32),
        compiler_params=pltpu.CompilerParams(use_tc_tiling_on_sc=False),
        scratch_types=[
            pltpu.VMEM((_CHUNK // 128, 128), jnp.int32),
            pltpu.VMEM((_CHUNK, 16), jnp.float32),
            pltpu.SemaphoreType.DMA,
            pltpu.SemaphoreType.DMA,
        ],
    )
    def sc_compact(dest_hbm, data_hbm, out_hbm, idx_v, data_v, sem_in, sem):
        wid = lax.axis_index("s") * 2 + lax.axis_index("c")
        base = wid * _CHUNK
        in1 = pltpu.async_copy(dest_hbm.at[wid], idx_v, sem_in)
        in2 = pltpu.async_copy(data_hbm.at[pl.ds(base, _CHUNK)], data_v,
                               sem_in)
        in1.wait()
        in2.wait()
        descs = []
        for j in range(_CHUNK // 128):
            descs.append(pltpu.async_copy(
                data_v.at[pl.ds(j * 128, 128)],
                out_hbm.at[idx_v.at[j]],
                sem))
        for d in descs:
            d.wait()

    return sc_compact(dest3, packed)


def _prep(v):
    flat = v.T.reshape(-1)
    return jnp.pad(flat, (0, _NPAD - _N)).reshape(_ROWS, 128)


def kernel(cls_logits, bbox_preds, anchors):
    cl = cls_logits.reshape(_A, 2, _H * _W)
    bb = bbox_preds.reshape(_A, 4, _H * _W)
    l0 = _prep(cl[:, 0, :])
    l1 = _prep(cl[:, 1, :])
    d = [_prep(bb[:, k, :]) for k in range(4)]
    a = [jnp.pad(anchors[:, k], (0, _NPAD - _N)).reshape(_ROWS, 128)
         for k in range(4)]

    f32 = jnp.float32
    s, x1, y1, x2, y2, ar, dest = pl.pallas_call(
        _stage1_body,
        out_shape=(
            jax.ShapeDtypeStruct((_ROWS, 128), f32),
            jax.ShapeDtypeStruct((_ROWS, 128), f32),
            jax.ShapeDtypeStruct((_ROWS, 128), f32),
            jax.ShapeDtypeStruct((_ROWS, 128), f32),
            jax.ShapeDtypeStruct((_ROWS, 128), f32),
            jax.ShapeDtypeStruct((_ROWS, 128), f32),
            jax.ShapeDtypeStruct((_ROWS, 128), jnp.int32),
        ),
    )(l0, l1, *d, *a)

    packed = jnp.stack(
        [v.reshape(-1) for v in (s, x1, y1, x2, y2, ar)], axis=-1)
    packed = jnp.pad(packed, ((0, 0), (0, 10)))
    dest3 = dest.reshape(_NTILES, _CHUNK // 128, 128)

    compact = _sc_compact_call(dest3, packed)

    dat = compact.T.reshape(16, _SROWS, 128)
    cols = [compact[:, k].reshape(_SROWS, 128) for k in range(1, 6)]
    out = pl.pallas_call(
        _stage3_body,
        in_specs=[
            pl.BlockSpec(memory_space=pltpu.VMEM),
            pl.BlockSpec(memory_space=pltpu.SMEM),
            pl.BlockSpec(memory_space=pltpu.SMEM),
            pl.BlockSpec(memory_space=pltpu.SMEM),
            pl.BlockSpec(memory_space=pltpu.SMEM),
            pl.BlockSpec(memory_space=pltpu.SMEM),
        ],
        out_shape=jax.ShapeDtypeStruct((_OUT_ROWS, 128), f32),
        scratch_shapes=[pltpu.VMEM((_SROWS, 128), f32)],
    )(dat, *cols)
    return out[:_POST, :4].reshape(1, _POST, 4)


# speedup vs baseline: 35.0383x; 1.2552x over previous
import functools

import jax
import jax.numpy as jnp
from jax import lax
from jax.experimental import pallas as pl
from jax.experimental.pallas import tpu as pltpu
from jax.experimental.pallas import tpu_sc as plsc

_A = 9
_H = 50
_W = 50
_N = _H * _W * _A
_ROWS = 192
_NPAD = _ROWS * 128
_PRE = 6000
_POST = 2000
_NSEL = 6144
_SROWS = 48
_TRASH = 6100
_OUT_ROWS = 2048
_NTILES = 32
_CHUNK = _NPAD // _NTILES


def _stage1_body(l0, l1, d0, d1, d2, d3, a0, a1, a2, a3,
                 s_o, x1_o, y1_o, x2_o, y2_o, ar_o, dest_o):
    f32 = jnp.float32
    pos = (lax.broadcasted_iota(jnp.int32, (_ROWS, 128), 0) * 128
           + lax.broadcasted_iota(jnp.int32, (_ROWS, 128), 1))
    real = pos < _N

    l0v = l0[...]
    l1v = l1[...]
    mx = jnp.maximum(l0v, l1v)
    e0 = jnp.exp(l0v - mx)
    e1 = jnp.exp(l1v - mx)
    s = e1 / (e0 + e1)
    s_o[...] = s

    w = a2[...] - a0[...]
    h = a3[...] - a1[...]
    cx = a0[...] + 0.5 * w
    cy = a1[...] + 0.5 * h
    pcx = d0[...] * w + cx
    pcy = d1[...] * h + cy
    pw = jnp.exp(jnp.clip(d2[...], -4.0, 4.0)) * w
    ph = jnp.exp(jnp.clip(d3[...], -4.0, 4.0)) * h
    x1 = pcx - 0.5 * pw
    y1 = pcy - 0.5 * ph
    x2 = pcx + 0.5 * pw
    y2 = pcy + 0.5 * ph
    x1_o[...] = x1
    y1_o[...] = y1
    x2_o[...] = x2
    y2_o[...] = y2
    ar_o[...] = (x2 - x1) * (y2 - y1)

    key = jnp.where(real, lax.bitcast_convert_type(s, jnp.int32),
                    jnp.int32(-1))
    vstar = jnp.int32(0)
    for b in range(30, -1, -1):
        t = vstar | jnp.int32(1 << b)
        cnt = jnp.sum((key >= t).astype(jnp.int32))
        vstar = jnp.where(cnt >= _PRE, t, vstar)
    n_gt = jnp.sum((key > vstar).astype(jnp.int32))
    need = _PRE - n_gt
    tie = key == vstar
    cc = jnp.int32(0)
    for b in range(14, -1, -1):
        t = cc | jnp.int32(1 << b)
        cnt = jnp.sum((tie & (pos < t)).astype(jnp.int32))
        cc = jnp.where(cnt < need, t, cc)
    cstar = jnp.where(need > 0, cc + 1, jnp.int32(0))
    mask = (key > vstar) | (tie & (pos < cstar))

    mf = mask.astype(f32)
    tri_u = (lax.broadcasted_iota(jnp.int32, (128, 128), 0)
             <= lax.broadcasted_iota(jnp.int32, (128, 128), 1)).astype(f32)
    rowcum = jnp.dot(mf, tri_u, preferred_element_type=f32)
    totals = lax.broadcast_in_dim(rowcum[:, 127], (_ROWS, 128), (0,))
    tri_l = (lax.broadcasted_iota(jnp.int32, (_ROWS, _ROWS), 1)
             < lax.broadcasted_iota(jnp.int32, (_ROWS, _ROWS), 0)).astype(f32)
    offs = jnp.dot(tri_l, totals, preferred_element_type=f32)
    cum = rowcum + offs
    dest_o[...] = jnp.where(mask, cum.astype(jnp.int32) - 1,
                            jnp.int32(_TRASH))


def _stage3_body(dat_ref, x1s, y1s, x2s, y2s, ars, out_ref):
    f32 = jnp.float32
    s = dat_ref[0]
    x1 = dat_ref[1]
    y1 = dat_ref[2]
    x2 = dat_ref[3]
    y2 = dat_ref[4]
    area = dat_ref[5]
    pos = (lax.broadcasted_iota(jnp.int32, (_SROWS, 128), 0) * 128
           + lax.broadcasted_iota(jnp.int32, (_SROWS, 128), 1))
    out_ref[...] = jnp.zeros((_OUT_ROWS, 128), f32)
    ci = lax.broadcasted_iota(jnp.int32, (1, 128), 1)

    ls0 = jnp.where(pos < _PRE, s, -1.0)
    am0 = jnp.argmax(ls0).astype(jnp.int32)
    mv0 = jnp.max(ls0)

    def body(k, st):
        ls, am, mv = st
        valid = mv >= 0.0
        r = lax.div(am, jnp.int32(128))
        c = lax.rem(am, jnp.int32(128))
        bx1 = x1s[r, c]
        by1 = y1s[r, c]
        bx2 = x2s[r, c]
        by2 = y2s[r, c]
        bar = ars[r, c]
        xx1 = jnp.maximum(bx1, x1)
        yy1 = jnp.maximum(by1, y1)
        xx2 = jnp.minimum(bx2, x2)
        yy2 = jnp.minimum(by2, y2)
        inter = (jnp.maximum(xx2 - xx1, 0.0)
                 * jnp.maximum(yy2 - yy1, 0.0))
        iou = inter / (bar + area - inter + 1e-9)
        supp = (iou > 0.7) | (pos == am)
        ls2 = jnp.where(valid & supp, -1.0, ls)

        @pl.when(valid)
        def _():
            row = jnp.where(ci == 0, bx1,
                            jnp.where(ci == 1, by1,
                                      jnp.where(ci == 2, bx2,
                                                jnp.where(ci == 3, by2,
                                                          0.0))))
            out_ref[pl.ds(k, 1), :] = row

        am2 = jnp.argmax(ls2).astype(jnp.int32)
        mv2 = jnp.max(ls2)
        return (ls2, am2, mv2)

    lax.fori_loop(0, _POST, body, (ls0, am0, mv0))


def _sc_compact_call(dest3, packed):
    mesh = plsc.VectorSubcoreMesh(core_axis_name="c", subcore_axis_name="s")

    @functools.partial(
        pl.kernel,
        mesh=mesh,
        out_type=jax.ShapeDtypeStruct((_NSEL, 16), jnp.float32),
        compiler_params=pltpu.CompilerParams(use_tc_tiling_on_sc=False),
        scratch_types=[
            pltpu.VMEM((_CHUNK // 128, 128), jnp.int32),
            pltpu.VMEM((_CHUNK, 16), jnp.float32),
            pltpu.SemaphoreType.DMA,
            pltpu.SemaphoreType.DMA,
        ],
    )
    def sc_compact(dest_hbm, data_hbm, out_hbm, idx_v, data_v, sem_in, sem):
        wid = lax.axis_index("s") * 2 + lax.axis_index("c")
        base = wid * _CHUNK
        in1 = pltpu.async_copy(dest_hbm.at[wid], idx_v, sem_in)
        in2 = pltpu.async_copy(data_hbm.at[pl.ds(base, _CHUNK)], data_v,
                               sem_in)
        in1.wait()
        in2.wait()
        descs = []
        for j in range(_CHUNK // 128):
            descs.append(pltpu.async_copy(
                data_v.at[pl.ds(j * 128, 128)],
                out_hbm.at[idx_v.at[j]],
                sem))
        for d in descs:
            d.wait()

    return sc_compact(dest3, packed)


def _prep(v):
    flat = v.T.reshape(-1)
    return jnp.pad(flat, (0, _NPAD - _N)).reshape(_ROWS, 128)


def kernel(cls_logits, bbox_preds, anchors):
    cl = cls_logits.reshape(_A, 2, _H * _W)
    bb = bbox_preds.reshape(_A, 4, _H * _W)
    l0 = _prep(cl[:, 0, :])
    l1 = _prep(cl[:, 1, :])
    d = [_prep(bb[:, k, :]) for k in range(4)]
    a = [jnp.pad(anchors[:, k], (0, _NPAD - _N)).reshape(_ROWS, 128)
         for k in range(4)]

    f32 = jnp.float32
    s, x1, y1, x2, y2, ar, dest = pl.pallas_call(
        _stage1_body,
        out_shape=(
            jax.ShapeDtypeStruct((_ROWS, 128), f32),
            jax.ShapeDtypeStruct((_ROWS, 128), f32),
            jax.ShapeDtypeStruct((_ROWS, 128), f32),
            jax.ShapeDtypeStruct((_ROWS, 128), f32),
            jax.ShapeDtypeStruct((_ROWS, 128), f32),
            jax.ShapeDtypeStruct((_ROWS, 128), f32),
            jax.ShapeDtypeStruct((_ROWS, 128), jnp.int32),
        ),
    )(l0, l1, *d, *a)

    packed = jnp.stack(
        [v.reshape(-1) for v in (s, x1, y1, x2, y2, ar)], axis=-1)
    packed = jnp.pad(packed, ((0, 0), (0, 10)))
    dest3 = dest.reshape(_NTILES, _CHUNK // 128, 128)

    compact = _sc_compact_call(dest3, packed)

    dat = compact.T.reshape(16, _SROWS, 128)
    cols = [compact[:, k].reshape(_SROWS, 128) for k in range(1, 6)]
    out = pl.pallas_call(
        _stage3_body,
        in_specs=[
            pl.BlockSpec(memory_space=pltpu.VMEM),
            pl.BlockSpec(memory_space=pltpu.SMEM),
            pl.BlockSpec(memory_space=pltpu.SMEM),
            pl.BlockSpec(memory_space=pltpu.SMEM),
            pl.BlockSpec(memory_space=pltpu.SMEM),
            pl.BlockSpec(memory_space=pltpu.SMEM),
        ],
        out_shape=jax.ShapeDtypeStruct((_OUT_ROWS, 128), f32),
    )(dat, *cols)
    return out[:_POST, :4].reshape(1, _POST, 4)


# speedup vs baseline: 38.2988x; 1.0931x over previous
import functools

import jax
import jax.numpy as jnp
from jax import lax
from jax.experimental import pallas as pl
from jax.experimental.pallas import tpu as pltpu
from jax.experimental.pallas import tpu_sc as plsc

_A = 9
_H = 50
_W = 50
_N = _H * _W * _A
_ROWS = 192
_NPAD = _ROWS * 128
_PRE = 6000
_POST = 2000
_NSEL = 6144
_SROWS = 48
_TRASH = 6100
_OUT_ROWS = 2048
_NTILES = 32
_CHUNK = _NPAD // _NTILES


def _stage1_body(l0, l1, d0, d1, d2, d3, a0, a1, a2, a3,
                 s_o, x1_o, y1_o, x2_o, y2_o, ar_o, dest_o):
    f32 = jnp.float32
    pos = (lax.broadcasted_iota(jnp.int32, (_ROWS, 128), 0) * 128
           + lax.broadcasted_iota(jnp.int32, (_ROWS, 128), 1))
    real = pos < _N

    l0v = l0[...]
    l1v = l1[...]
    mx = jnp.maximum(l0v, l1v)
    e0 = jnp.exp(l0v - mx)
    e1 = jnp.exp(l1v - mx)
    s = e1 / (e0 + e1)
    s_o[...] = s

    w = a2[...] - a0[...]
    h = a3[...] - a1[...]
    cx = a0[...] + 0.5 * w
    cy = a1[...] + 0.5 * h
    pcx = d0[...] * w + cx
    pcy = d1[...] * h + cy
    pw = jnp.exp(jnp.clip(d2[...], -4.0, 4.0)) * w
    ph = jnp.exp(jnp.clip(d3[...], -4.0, 4.0)) * h
    x1 = pcx - 0.5 * pw
    y1 = pcy - 0.5 * ph
    x2 = pcx + 0.5 * pw
    y2 = pcy + 0.5 * ph
    x1_o[...] = x1
    y1_o[...] = y1
    x2_o[...] = x2
    y2_o[...] = y2
    ar_o[...] = (x2 - x1) * (y2 - y1)

    key = jnp.where(real, lax.bitcast_convert_type(s, jnp.int32),
                    jnp.int32(-1))
    vstar = jnp.int32(0)
    for b in range(30, -1, -1):
        t = vstar | jnp.int32(1 << b)
        cnt = jnp.sum((key >= t).astype(jnp.int32))
        vstar = jnp.where(cnt >= _PRE, t, vstar)
    n_gt = jnp.sum((key > vstar).astype(jnp.int32))
    need = _PRE - n_gt
    tie = key == vstar
    cc = jnp.int32(0)
    for b in range(14, -1, -1):
        t = cc | jnp.int32(1 << b)
        cnt = jnp.sum((tie & (pos < t)).astype(jnp.int32))
        cc = jnp.where(cnt < need, t, cc)
    cstar = jnp.where(need > 0, cc + 1, jnp.int32(0))
    mask = (key > vstar) | (tie & (pos < cstar))

    mf = mask.astype(f32)
    tri_u = (lax.broadcasted_iota(jnp.int32, (128, 128), 0)
             <= lax.broadcasted_iota(jnp.int32, (128, 128), 1)).astype(f32)
    rowcum = jnp.dot(mf, tri_u, preferred_element_type=f32)
    totals = lax.broadcast_in_dim(rowcum[:, 127], (_ROWS, 128), (0,))
    tri_l = (lax.broadcasted_iota(jnp.int32, (_ROWS, _ROWS), 1)
             < lax.broadcasted_iota(jnp.int32, (_ROWS, _ROWS), 0)).astype(f32)
    offs = jnp.dot(tri_l, totals, preferred_element_type=f32)
    cum = rowcum + offs
    dest_o[...] = jnp.where(mask, cum.astype(jnp.int32) - 1,
                            jnp.int32(_TRASH))


_BROWS = 64


def _sort_stage(arrs, j):
    if j < 128:
        cbit = (lax.broadcasted_iota(jnp.int32, (_BROWS, 128), 1)
                & jnp.int32(j)) == 0
        parts = [jnp.where(cbit, jnp.roll(a, -j, axis=1),
                           jnp.roll(a, j, axis=1)) for a in arrs]
    else:
        jr = j // 128
        cbit = (lax.broadcasted_iota(jnp.int32, (_BROWS, 128), 0)
                & jnp.int32(jr)) == 0
        parts = [jnp.where(cbit, jnp.roll(a, -jr, axis=0),
                           jnp.roll(a, jr, axis=0)) for a in arrs]
    return cbit, parts


def _stage2b_body(dat_ref, xo1, yo1, xo2, yo2, aro):
    i32 = jnp.int32
    posb = (lax.broadcasted_iota(i32, (_BROWS, 128), 0) * 128
            + lax.broadcasted_iota(i32, (_BROWS, 128), 1))
    zpad = jnp.zeros((_BROWS - _SROWS, 128), jnp.float32)

    def padded(v):
        return jnp.concatenate([v, zpad], axis=0)

    skey = lax.bitcast_convert_type(dat_ref[0], i32)
    ipad = jnp.zeros((_BROWS - _SROWS, 128), i32)
    key = jnp.where(posb < _PRE,
                    jnp.concatenate([skey, ipad], axis=0), i32(-1))
    pid = posb
    pay = [padded(dat_ref[k]) for k in range(1, 6)]

    arrs = [key, pid] + pay
    n = _BROWS * 128
    k = 2
    while k <= n:
        j = k // 2
        while j >= 1:
            cbit, parts = _sort_stage(arrs, j)
            kk, pp = arrs[0], arrs[1]
            pk, ppid = parts[0], parts[1]
            if k >= 128:
                dirbit = (lax.broadcasted_iota(i32, (_BROWS, 128), 0)
                          & i32(k // 128)) == 0
            else:
                dirbit = (lax.broadcasted_iota(i32, (_BROWS, 128), 1)
                          & i32(k)) == 0
            less = (kk > pk) | ((kk == pk) & (pp < ppid))
            take_mine = (cbit == dirbit) == less
            arrs = [jnp.where(take_mine, a, p)
                    for a, p in zip(arrs, parts)]
            j //= 2
        k *= 2

    xo1[...] = arrs[2][:_SROWS]
    yo1[...] = arrs[3][:_SROWS]
    xo2[...] = arrs[4][:_SROWS]
    yo2[...] = arrs[5][:_SROWS]
    aro[...] = arrs[6][:_SROWS]


def _stage3_body(x1r, y1r, x2r, y2r, arr,
                 x1s, y1s, x2s, y2s, ars, out_ref):
    f32 = jnp.float32
    i32 = jnp.int32
    x1 = x1r[...]
    y1 = y1r[...]
    x2 = x2r[...]
    y2 = y2r[...]
    area = arr[...]
    pos = (lax.broadcasted_iota(i32, (_SROWS, 128), 0) * 128
           + lax.broadcasted_iota(i32, (_SROWS, 128), 1))
    out_ref[...] = jnp.zeros((_OUT_ROWS, 128), f32)
    ci = lax.broadcasted_iota(i32, (1, 128), 1)

    lsp0 = jnp.where(pos < _PRE, pos, i32(_NSEL))
    pk0 = jnp.min(lsp0)

    def body(k, st):
        lsp, pick = st
        valid = pick < _PRE
        safe = jnp.minimum(pick, i32(_NSEL - 1))
        r = lax.div(safe, i32(128))
        c = lax.rem(safe, i32(128))
        bx1 = x1s[r, c]
        by1 = y1s[r, c]
        bx2 = x2s[r, c]
        by2 = y2s[r, c]
        bar = ars[r, c]
        xx1 = jnp.maximum(bx1, x1)
        yy1 = jnp.maximum(by1, y1)
        xx2 = jnp.minimum(bx2, x2)
        yy2 = jnp.minimum(by2, y2)
        inter = (jnp.maximum(xx2 - xx1, 0.0)
                 * jnp.maximum(yy2 - yy1, 0.0))
        iou = inter / (bar + area - inter + 1e-9)
        supp = (iou > 0.7) | (pos == pick)
        lsp2 = jnp.where(valid & supp, i32(_NSEL), lsp)

        @pl.when(valid)
        def _():
            row = jnp.where(ci == 0, bx1,
                            jnp.where(ci == 1, by1,
                                      jnp.where(ci == 2, bx2,
                                                jnp.where(ci == 3, by2,
                                                          0.0))))
            out_ref[pl.ds(k, 1), :] = row

        return (lsp2, jnp.min(lsp2))

    lax.fori_loop(0, _POST, body, (lsp0, pk0))


def _sc_compact_call(dest3, packed):
    mesh = plsc.VectorSubcoreMesh(core_axis_name="c", subcore_axis_name="s")

    @functools.partial(
        pl.kernel,
        mesh=mesh,
        out_type=jax.ShapeDtypeStruct((_NSEL, 16), jnp.float32),
        compiler_params=pltpu.CompilerParams(use_tc_tiling_on_sc=False),
        scratch_types=[
            pltpu.VMEM((_CHUNK // 128, 128), jnp.int32),
            pltpu.VMEM((_CHUNK, 16), jnp.float32),
            pltpu.SemaphoreType.DMA,
            pltpu.SemaphoreType.DMA,
        ],
    )
    def sc_compact(dest_hbm, data_hbm, out_hbm, idx_v, data_v, sem_in, sem):
        wid = lax.axis_index("s") * 2 + lax.axis_index("c")
        base = wid * _CHUNK
        in1 = pltpu.async_copy(dest_hbm.at[wid], idx_v, sem_in)
        in2 = pltpu.async_copy(data_hbm.at[pl.ds(base, _CHUNK)], data_v,
                               sem_in)
        in1.wait()
        in2.wait()
        descs = []
        for j in range(_CHUNK // 128):
            descs.append(pltpu.async_copy(
                data_v.at[pl.ds(j * 128, 128)],
                out_hbm.at[idx_v.at[j]],
                sem))
        for d in descs:
            d.wait()

    return sc_compact(dest3, packed)


def _prep(v):
    flat = v.T.reshape(-1)
    return jnp.pad(flat, (0, _NPAD - _N)).reshape(_ROWS, 128)


def kernel(cls_logits, bbox_preds, anchors):
    cl = cls_logits.reshape(_A, 2, _H * _W)
    bb = bbox_preds.reshape(_A, 4, _H * _W)
    l0 = _prep(cl[:, 0, :])
    l1 = _prep(cl[:, 1, :])
    d = [_prep(bb[:, k, :]) for k in range(4)]
    a = [jnp.pad(anchors[:, k], (0, _NPAD - _N)).reshape(_ROWS, 128)
         for k in range(4)]

    f32 = jnp.float32
    s, x1, y1, x2, y2, ar, dest = pl.pallas_call(
        _stage1_body,
        out_shape=(
            jax.ShapeDtypeStruct((_ROWS, 128), f32),
            jax.ShapeDtypeStruct((_ROWS, 128), f32),
            jax.ShapeDtypeStruct((_ROWS, 128), f32),
            jax.ShapeDtypeStruct((_ROWS, 128), f32),
            jax.ShapeDtypeStruct((_ROWS, 128), f32),
            jax.ShapeDtypeStruct((_ROWS, 128), f32),
            jax.ShapeDtypeStruct((_ROWS, 128), jnp.int32),
        ),
    )(l0, l1, *d, *a)

    packed = jnp.stack(
        [v.reshape(-1) for v in (s, x1, y1, x2, y2, ar)], axis=-1)
    packed = jnp.pad(packed, ((0, 0), (0, 10)))
    dest3 = dest.reshape(_NTILES, _CHUNK // 128, 128)

    compact = _sc_compact_call(dest3, packed)

    dat = compact.T.reshape(16, _SROWS, 128)
    srt = pl.pallas_call(
        _stage2b_body,
        out_shape=tuple(jax.ShapeDtypeStruct((_SROWS, 128), f32)
                        for _ in range(5)),
    )(dat)

    out = pl.pallas_call(
        _stage3_body,
        in_specs=[pl.BlockSpec(memory_space=pltpu.VMEM)] * 5
        + [pl.BlockSpec(memory_space=pltpu.SMEM)] * 5,
        out_shape=jax.ShapeDtypeStruct((_OUT_ROWS, 128), f32),
    )(*srt, *srt)
    return out[:_POST, :4].reshape(1, _POST, 4)
